# baseline (speedup 1.0000x reference)
import functools

import jax
import jax.numpy as jnp
import numpy as np
from jax import lax
from jax.experimental import pallas as pl
from jax.experimental.pallas import tpu as pltpu
from jax.experimental.pallas import tpu_sc as plsc

_D = 64
_B = 16384
_H = 50
_TOTAL = _B * _H
_NC, _NS = 2, 16
_NW = _NC * _NS
_BPW = _B // _NW
_NBT = _BPW // 128
_PER_W = _BPW * _H
_NUNIT = _H * _NBT

_K = 2.0
_IR = np.float32(2.0 * _K / (1.0 + np.sqrt(1.0 + 4.0 * _K * _K)))
_EPS = np.float32(1e-05)


def _rsqrt(x):
    i = plsc.bitcast(x, jnp.int32)
    i = jnp.int32(0x5F3759DF) - (i >> 1)
    y = plsc.bitcast(i, jnp.float32)
    for _ in range(3):
        y = y * (jnp.float32(1.5) - jnp.float32(0.5) * x * y * y)
    return y


def _soft_clip_scale(sumsq):
    y = _rsqrt(sumsq)
    norm = sumsq * y
    one = jnp.full((16,), 1.0, jnp.float32)
    return jnp.where(norm <= _IR, _IR * y,
                     jnp.where(norm >= jnp.float32(1.0), (jnp.float32(1.0) - _EPS) * y,
                               one))


def _embed_body(idx_hbm, table_hbm, out_hbm, idx_v,
                idxh0, idxh1, in_v0, in_v1, out_t0, out_t1,
                sem_in0, sem_in1, sem_out0, sem_out1):
    wid = lax.axis_index("s") * _NC + lax.axis_index("c")
    base = wid * _PER_W
    bt_base = wid * _NBT
    pltpu.sync_copy(idx_hbm.at[pl.ds(base, _PER_W)], idx_v)
    idxhs = (idxh0, idxh1)
    in_bufs = (in_v0, in_v1)
    out_bufs = (out_t0, out_t1)
    sem_ins = (sem_in0, sem_in1)
    sem_outs = (sem_out0, sem_out1)
    lane = lax.iota(jnp.int32, 16)
    sc_idx = [((2 * k) + (lane >> 3)) * 1024 + (lane & 7) * 128
              for k in range(_D // 16)]

    def build_idx(u, b):
        h = u // _NBT
        bt = u % _NBT
        for g in range(8):
            pos = jnp.full((16,), (bt * 128 + g * 16) * _H, jnp.int32) + lane * _H + h
            idxhs[b][pl.ds(g * 16, 16)] = plsc.load_gather(idx_v, [pos])

    def gather_desc(b):
        return pltpu.make_async_copy(table_hbm.at[idxhs[b]], in_bufs[b], sem_ins[b])

    def put_descs(u, b):
        h = u // _NBT
        btg = bt_base + u % _NBT
        return [pltpu.make_async_copy(out_bufs[b].at[pl.ds(dt * 1024, 1024)],
                                      out_hbm.at[h, dt, btg], sem_outs[b])
                for dt in range(_D // 8)]

    def compute(b):
        in_v, out_t = in_bufs[b], out_bufs[b]

        def quad_body(q, carry2):
            for i in range(4):
                r = q * 4 + i
                v = [in_v[r, pl.ds(k * 16, 16)] for k in range(_D // 16)]
                sq = v[0] * v[0] + v[1] * v[1] + v[2] * v[2] + v[3] * v[3]
                cs = plsc.cumsum(sq)
                tot = jnp.full((16,), cs[15], jnp.float32)
                scale = _soft_clip_scale(tot)
                rvec = jnp.full((16,), r, jnp.int32)
                for k in range(_D // 16):
                    plsc.store_scatter(out_t, [sc_idx[k] + rvec],
                                       (v[k] + jnp.float32(1e-15)) * scale)
            return carry2

        lax.fori_loop(0, 32, quad_body, 0, unroll=False)

    build_idx(0, 0)
    gather_desc(0).start()
    build_idx(1, 1)
    gather_desc(1).start()

    def pair_body(p, carry):
        u0 = p * 2
        for b in range(2):
            u = u0 + b
            gather_desc(b).wait()

            @pl.when(u0 > 0)
            def _wait_out():
                for d in put_descs(u, b):
                    d.wait()

            compute(b)
            for d in put_descs(u, b):
                d.start()

            @pl.when(u < _NUNIT - 2)
            def _next_gather():
                build_idx(u + 2, b)
                gather_desc(b).start()

        return carry

    lax.fori_loop(0, _NUNIT // 2, pair_body, 0, unroll=False)
    for b in range(2):
        for d in put_descs(_NUNIT - 2 + b, b):
            d.wait()


_embed_sc = functools.partial(
    pl.kernel,
    out_type=jax.ShapeDtypeStruct((_H, _D // 8, _B // 128, 1024), jnp.float32),
    mesh=plsc.VectorSubcoreMesh(core_axis_name="c", subcore_axis_name="s"),
    compiler_params=pltpu.CompilerParams(needs_layout_passes=False,
                                         use_tc_tiling_on_sc=False),
    scratch_types=[
        pltpu.VMEM((_PER_W,), jnp.int32),
        pltpu.VMEM((128,), jnp.int32),
        pltpu.VMEM((128,), jnp.int32),
        pltpu.VMEM((128, _D), jnp.float32),
        pltpu.VMEM((128, _D), jnp.float32),
        pltpu.VMEM((_D // 8 * 1024,), jnp.float32),
        pltpu.VMEM((_D // 8 * 1024,), jnp.float32),
        pltpu.SemaphoreType.DMA,
        pltpu.SemaphoreType.DMA,
        pltpu.SemaphoreType.DMA,
        pltpu.SemaphoreType.DMA,
    ],
)(_embed_body)


def kernel(inputs, table):
    flat_idx = inputs.reshape(_TOTAL)
    out4 = _embed_sc(flat_idx, table)
    out5 = out4.reshape(_H, _D // 8, _B // 128, 8, 128)
    return out5.transpose(2, 4, 0, 1, 3).reshape(_B, _H, _D)


# speedup vs baseline: 1.0003x; 1.0003x over previous
import functools

import jax
import jax.numpy as jnp
import numpy as np
from jax import lax
from jax.experimental import pallas as pl
from jax.experimental.pallas import tpu as pltpu
from jax.experimental.pallas import tpu_sc as plsc

_D = 64
_B = 16384
_H = 50
_TOTAL = _B * _H
_NC, _NS = 2, 16
_NW = _NC * _NS
_BPW = _B // _NW
_NBT = _BPW // 128
_PER_W = _BPW * _H
_NUNIT = _H * _NBT

_K = 2.0
_IR = np.float32(2.0 * _K / (1.0 + np.sqrt(1.0 + 4.0 * _K * _K)))
_EPS = np.float32(1e-05)


def _rsqrt(x):
    i = plsc.bitcast(x, jnp.int32)
    i = jnp.int32(0x5F3759DF) - (i >> 1)
    y = plsc.bitcast(i, jnp.float32)
    for _ in range(3):
        y = y * (jnp.float32(1.5) - jnp.float32(0.5) * x * y * y)
    return y


def _soft_clip_scale(sumsq):
    y = _rsqrt(sumsq)
    norm = sumsq * y
    one = jnp.full((16,), 1.0, jnp.float32)
    return jnp.where(norm <= _IR, _IR * y,
                     jnp.where(norm >= jnp.float32(1.0), (jnp.float32(1.0) - _EPS) * y,
                               one))


def _embed_body(idx_hbm, table_hbm, out_hbm, idx_v,
                idxh0, idxh1, in_v0, in_v1, out_t0, out_t1,
                sem_in0, sem_in1, sem_out0, sem_out1):
    wid = lax.axis_index("s") * _NC + lax.axis_index("c")
    base = wid * _PER_W
    bt_base = wid * _NBT
    pltpu.sync_copy(idx_hbm.at[pl.ds(base, _PER_W)], idx_v)
    idxhs = (idxh0, idxh1)
    in_bufs = (in_v0, in_v1)
    out_bufs = (out_t0, out_t1)
    sem_ins = (sem_in0, sem_in1)
    sem_outs = (sem_out0, sem_out1)
    lane = lax.iota(jnp.int32, 16)
    sc_idx = [((2 * k) + (lane >> 3)) * 1024 + (lane & 7) * 128
              for k in range(_D // 16)]

    def build_idx(u, b):
        h = u // _NBT
        bt = u % _NBT
        for g in range(8):
            pos = jnp.full((16,), (bt * 128 + g * 16) * _H, jnp.int32) + lane * _H + h
            idxhs[b][pl.ds(g * 16, 16)] = plsc.load_gather(idx_v, [pos])

    def gather_desc(b):
        return pltpu.make_async_copy(table_hbm.at[idxhs[b]], in_bufs[b], sem_ins[b])

    def put_descs(u, b):
        h = u // _NBT
        btg = bt_base + u % _NBT
        return [pltpu.make_async_copy(out_bufs[b].at[pl.ds(dt * 1024, 1024)],
                                      out_hbm.at[h, dt, btg], sem_outs[b])
                for dt in range(_D // 8)]

    def compute(b):
        in_v, out_t = in_bufs[b], out_bufs[b]

        def quad_body(q, carry2):
            for i in range(4):
                r = q * 4 + i
                v = [in_v[r, pl.ds(k * 16, 16)] for k in range(_D // 16)]
                sq = v[0] * v[0] + v[1] * v[1] + v[2] * v[2] + v[3] * v[3]
                cs = plsc.cumsum(sq)
                tot = jnp.full((16,), cs[15], jnp.float32)
                scale = _soft_clip_scale(tot)
                rvec = jnp.full((16,), r, jnp.int32)
                for k in range(_D // 16):
                    plsc.store_scatter(out_t, [sc_idx[k] + rvec],
                                       (v[k] + jnp.float32(1e-15)) * scale)
            return carry2

        lax.fori_loop(0, 32, quad_body, 0, unroll=False)

    build_idx(0, 0)
    gather_desc(0).start()
    build_idx(1, 1)
    gather_desc(1).start()

    def pair_body(p, carry):
        u0 = p * 2
        for b in range(2):
            u = u0 + b
            gather_desc(b).wait()

            @pl.when(u0 > 0)
            def _wait_out():
                for d in put_descs(u, b):
                    d.wait()

            compute(b)
            for d in put_descs(u, b):
                d.start()

            @pl.when(u < _NUNIT - 2)
            def _next_gather():
                build_idx(u + 2, b)
                gather_desc(b).start()

        return carry

    lax.fori_loop(0, _NUNIT // 2, pair_body, 0, unroll=False)
    for b in range(2):
        for d in put_descs(_NUNIT - 2 + b, b):
            d.wait()


_embed_sc = functools.partial(
    pl.kernel,
    out_type=jax.ShapeDtypeStruct((_H, _D // 8, _B // 128, 1024), jnp.float32),
    mesh=plsc.VectorSubcoreMesh(core_axis_name="c", subcore_axis_name="s"),
    compiler_params=pltpu.CompilerParams(needs_layout_passes=False,
                                         use_tc_tiling_on_sc=False,
                                         disable_bounds_checks=True),
    scratch_types=[
        pltpu.VMEM((_PER_W,), jnp.int32),
        pltpu.VMEM((128,), jnp.int32),
        pltpu.VMEM((128,), jnp.int32),
        pltpu.VMEM((128, _D), jnp.float32),
        pltpu.VMEM((128, _D), jnp.float32),
        pltpu.VMEM((_D // 8 * 1024,), jnp.float32),
        pltpu.VMEM((_D // 8 * 1024,), jnp.float32),
        pltpu.SemaphoreType.DMA,
        pltpu.SemaphoreType.DMA,
        pltpu.SemaphoreType.DMA,
        pltpu.SemaphoreType.DMA,
    ],
)(_embed_body)


def kernel(inputs, table):
    flat_idx = inputs.reshape(_TOTAL)
    out4 = _embed_sc(flat_idx, table)
    out5 = out4.reshape(_H, _D // 8, _B // 128, 8, 128)
    return out5.transpose(2, 4, 0, 1, 3).reshape(_B, _H, _D)


# speedup vs baseline: 1.4032x; 1.4028x over previous
import functools

import jax
import jax.numpy as jnp
import numpy as np
from jax import lax
from jax.experimental import pallas as pl
from jax.experimental.pallas import tpu as pltpu
from jax.experimental.pallas import tpu_sc as plsc

_D = 64
_B = 16384
_H = 50
_TOTAL = _B * _H
_NC, _NS = 2, 16
_NW = _NC * _NS
_BPW = _B // _NW
_NBT = _BPW // 128
_PER_W = _BPW * _H
_NUNIT = _H * _NBT

_K = 2.0
_IR = np.float32(2.0 * _K / (1.0 + np.sqrt(1.0 + 4.0 * _K * _K)))
_EPS = np.float32(1e-05)


def _rsqrt(x):
    i = plsc.bitcast(x, jnp.int32)
    i = jnp.int32(0x5F3759DF) - (i >> 1)
    y = plsc.bitcast(i, jnp.float32)
    for _ in range(3):
        y = y * (jnp.float32(1.5) - jnp.float32(0.5) * x * y * y)
    return y


def _soft_clip_scale(sumsq):
    y = _rsqrt(sumsq)
    norm = sumsq * y
    one = jnp.full((16,), 1.0, jnp.float32)
    return jnp.where(norm <= _IR, _IR * y,
                     jnp.where(norm >= jnp.float32(1.0), (jnp.float32(1.0) - _EPS) * y,
                               one))


def _embed_body(idx_hbm, table_hbm, out_hbm, idx_v,
                idxh0, idxh1, in_v0, in_v1, out_t0, out_t1,
                sem_in0, sem_in1, sem_out0, sem_out1):
    wid = lax.axis_index("s") * _NC + lax.axis_index("c")
    base = wid * _PER_W
    bt_base = wid * _NBT
    pltpu.sync_copy(idx_hbm.at[pl.ds(base, _PER_W)], idx_v)
    idxhs = (idxh0, idxh1)
    in_bufs = (in_v0, in_v1)
    out_bufs = (out_t0, out_t1)
    sem_ins = (sem_in0, sem_in1)
    sem_outs = (sem_out0, sem_out1)
    lane = lax.iota(jnp.int32, 16)
    d_idx = [16 * k + lane for k in range(_D // 16)]

    def build_idx(u, b):
        h = u // _NBT
        bt = u % _NBT
        for g in range(8):
            pos = jnp.full((16,), (bt * 128 + g * 16) * _H, jnp.int32) + lane * _H + h
            idxhs[b][pl.ds(g * 16, 16)] = plsc.load_gather(idx_v, [pos])

    def gather_desc(b):
        return pltpu.make_async_copy(table_hbm.at[idxhs[b]], in_bufs[b], sem_ins[b])

    def put_descs(u, b):
        h = u // _NBT
        btg = bt_base + u % _NBT
        return [pltpu.make_async_copy(
                    out_bufs[b].at[pl.ds(dt * 8, 8), pl.ds(0, 128)],
                    out_hbm.at[h, dt, btg], sem_outs[b])
                for dt in range(_D // 8)]

    def compute(b):
        in_v, out_t = in_bufs[b], out_bufs[b]

        def quad_body(q, carry2):
            for i in range(4):
                r = q * 4 + i
                v = [in_v[r, pl.ds(k * 16, 16)] for k in range(_D // 16)]
                sq = v[0] * v[0] + v[1] * v[1] + v[2] * v[2] + v[3] * v[3]
                cs = plsc.cumsum(sq)
                tot = jnp.full((16,), cs[15], jnp.float32)
                scale = _soft_clip_scale(tot)
                rvec = jnp.full((16,), r, jnp.int32)
                for k in range(_D // 16):
                    plsc.store_scatter(out_t, [d_idx[k], rvec],
                                       (v[k] + jnp.float32(1e-15)) * scale)
            return carry2

        lax.fori_loop(0, 32, quad_body, 0, unroll=False)

    build_idx(0, 0)
    gather_desc(0).start()
    build_idx(1, 1)
    gather_desc(1).start()

    def pair_body(p, carry):
        u0 = p * 2
        for b in range(2):
            u = u0 + b
            gather_desc(b).wait()

            @pl.when(u0 > 0)
            def _wait_out():
                for d in put_descs(u, b):
                    d.wait()

            compute(b)
            for d in put_descs(u, b):
                d.start()

            @pl.when(u < _NUNIT - 2)
            def _next_gather():
                build_idx(u + 2, b)
                gather_desc(b).start()

        return carry

    lax.fori_loop(0, _NUNIT // 2, pair_body, 0, unroll=False)
    for b in range(2):
        for d in put_descs(_NUNIT - 2 + b, b):
            d.wait()


_embed_sc = functools.partial(
    pl.kernel,
    out_type=jax.ShapeDtypeStruct((_H, _D // 8, _B // 128, 8, 128), jnp.float32),
    mesh=plsc.VectorSubcoreMesh(core_axis_name="c", subcore_axis_name="s"),
    compiler_params=pltpu.CompilerParams(needs_layout_passes=False,
                                         use_tc_tiling_on_sc=False,
                                         disable_bounds_checks=True),
    scratch_types=[
        pltpu.VMEM((_PER_W,), jnp.int32),
        pltpu.VMEM((128,), jnp.int32),
        pltpu.VMEM((128,), jnp.int32),
        pltpu.VMEM((128, _D), jnp.float32),
        pltpu.VMEM((128, _D), jnp.float32),
        pltpu.VMEM((64, 129), jnp.float32),
        pltpu.VMEM((64, 129), jnp.float32),
        pltpu.SemaphoreType.DMA,
        pltpu.SemaphoreType.DMA,
        pltpu.SemaphoreType.DMA,
        pltpu.SemaphoreType.DMA,
    ],
)(_embed_body)


def kernel(inputs, table):
    flat_idx = inputs.reshape(_TOTAL)
    out5 = _embed_sc(flat_idx, table)
    return out5.transpose(2, 4, 0, 1, 3).reshape(_B, _H, _D)


# speedup vs baseline: 2.0313x; 1.4476x over previous
import functools

import jax
import jax.numpy as jnp
import numpy as np
from jax import lax
from jax.experimental import pallas as pl
from jax.experimental.pallas import tpu as pltpu
from jax.experimental.pallas import tpu_sc as plsc

_D = 64
_B = 16384
_H = 50
_TOTAL = _B * _H
_NC, _NS = 2, 16
_NW = _NC * _NS
_BPW = _B // _NW
_NBT = _BPW // 128
_PER_W = _BPW * _H
_NUNIT = _H * _NBT

_K = 2.0
_IR = np.float32(2.0 * _K / (1.0 + np.sqrt(1.0 + 4.0 * _K * _K)))
_EPS = np.float32(1e-05)


def _rsqrt(x):
    i = plsc.bitcast(x, jnp.int32)
    i = jnp.int32(0x5F3759DF) - (i >> 1)
    y = plsc.bitcast(i, jnp.float32)
    for _ in range(3):
        y = y * (jnp.float32(1.5) - jnp.float32(0.5) * x * y * y)
    return y


def _soft_clip_scale(sumsq):
    y = _rsqrt(sumsq)
    norm = sumsq * y
    one = jnp.full((16,), 1.0, jnp.float32)
    return jnp.where(norm <= _IR, _IR * y,
                     jnp.where(norm >= jnp.float32(1.0), (jnp.float32(1.0) - _EPS) * y,
                               one))


def _embed_body(idx_hbm, table_hbm, out_hbm, idx_v,
                idxh0, idxh1, in_v0, in_v1, out_t0, out_t1,
                sem_in0, sem_in1, sem_out0, sem_out1):
    wid = lax.axis_index("s") * _NC + lax.axis_index("c")
    base = wid * _PER_W
    bt_base = wid * _NBT
    pltpu.sync_copy(idx_hbm.at[pl.ds(base, _PER_W)], idx_v)
    idxhs = (idxh0, idxh1)
    in_bufs = (in_v0, in_v1)
    out_bufs = (out_t0, out_t1)
    sem_ins = (sem_in0, sem_in1)
    sem_outs = (sem_out0, sem_out1)
    lane = lax.iota(jnp.int32, 16)
    dperm = [(lane + m) & 15 for m in range(16)]

    def build_idx(u, b):
        h = u // _NBT
        bt = u % _NBT
        for g in range(8):
            pos = jnp.full((16,), (bt * 128 + g * 16) * _H, jnp.int32) + lane * _H + h
            idxhs[b][pl.ds(g * 16, 16)] = plsc.load_gather(idx_v, [pos])

    def gather_desc(b):
        return pltpu.make_async_copy(table_hbm.at[idxhs[b]], in_bufs[b], sem_ins[b])

    def put_descs(u, b):
        h = u // _NBT
        btg = bt_base + u % _NBT
        return [pltpu.make_async_copy(
                    out_bufs[b].at[pl.ds(dt * 8, 8)],
                    out_hbm.at[h, dt, btg], sem_outs[b])
                for dt in range(_D // 8)]

    def compute(b):
        in_v, out_t = in_bufs[b], out_bufs[b]

        def group_body(g, carry2):
            j0 = g * 16
            jrows = lane + jnp.full((16,), j0, jnp.int32)
            acc = jnp.zeros((16,), jnp.float32)
            for k in range(_D // 16):
                kofs = jnp.full((16,), 16 * k, jnp.int32)
                for m in range(16):
                    dcol = dperm[m] + kofs
                    val = plsc.load_gather(in_v, [jrows, dcol])
                    acc = acc + val * val
                    plsc.store_scatter(out_t, [dcol, jrows], val)
            scale = _soft_clip_scale(acc)
            for d in range(_D):
                t = out_t[d, pl.ds(j0, 16)]
                out_t[d, pl.ds(j0, 16)] = (t + jnp.float32(1e-15)) * scale
            return carry2

        lax.fori_loop(0, 8, group_body, 0, unroll=False)

    build_idx(0, 0)
    gather_desc(0).start()
    build_idx(1, 1)
    gather_desc(1).start()

    def pair_body(p, carry):
        u0 = p * 2
        for b in range(2):
            u = u0 + b
            gather_desc(b).wait()

            @pl.when(u0 > 0)
            def _wait_out():
                for d in put_descs(u, b):
                    d.wait()

            compute(b)
            for d in put_descs(u, b):
                d.start()

            @pl.when(u < _NUNIT - 2)
            def _next_gather():
                build_idx(u + 2, b)
                gather_desc(b).start()

        return carry

    lax.fori_loop(0, _NUNIT // 2, pair_body, 0, unroll=False)
    for b in range(2):
        for d in put_descs(_NUNIT - 2 + b, b):
            d.wait()


_embed_sc = functools.partial(
    pl.kernel,
    out_type=jax.ShapeDtypeStruct((_H, _D // 8, _B // 128, 8, 128), jnp.float32),
    mesh=plsc.VectorSubcoreMesh(core_axis_name="c", subcore_axis_name="s"),
    compiler_params=pltpu.CompilerParams(needs_layout_passes=False,
                                         use_tc_tiling_on_sc=False,
                                         disable_bounds_checks=True),
    scratch_types=[
        pltpu.VMEM((_PER_W,), jnp.int32),
        pltpu.VMEM((128,), jnp.int32),
        pltpu.VMEM((128,), jnp.int32),
        pltpu.VMEM((128, _D), jnp.float32),
        pltpu.VMEM((128, _D), jnp.float32),
        pltpu.VMEM((64, 128), jnp.float32),
        pltpu.VMEM((64, 128), jnp.float32),
        pltpu.SemaphoreType.DMA,
        pltpu.SemaphoreType.DMA,
        pltpu.SemaphoreType.DMA,
        pltpu.SemaphoreType.DMA,
    ],
)(_embed_body)


def kernel(inputs, table):
    flat_idx = inputs.reshape(_TOTAL)
    out5 = _embed_sc(flat_idx, table)
    return out5.transpose(2, 4, 0, 1, 3).reshape(_B, _H, _D)
